# baseline (speedup 1.0000x reference)
import math

import jax
import jax.numpy as jnp
from jax import lax
from jax.experimental import pallas as pl
from jax.experimental.pallas import tpu as pltpu

D = 768
L = 4096
BN = 2
NR = 5
TB = 256
TW = 2048
KF = TW + TB
_NEG = -3.0e38


def _hi_lo(x):
    hi = x.astype(jnp.bfloat16)
    lo = (x - hi.astype(jnp.float32)).astype(jnp.bfloat16)
    return hi, lo


def _proj_body(s_ref, t_ref, wq_ref, bq_ref, wk_ref, bk_ref,
               qdh_ref, qdl_ref, kh_ref, kl_ref, td_ref):
    s = s_ref[0]
    t = t_ref[0]
    dims = (((1,), (1,)), ((), ()))
    q = lax.dot_general(s, wq_ref[...], dims,
                        preferred_element_type=jnp.float32) + bq_ref[...]
    k = lax.dot_general(t, wk_ref[...], dims,
                        preferred_element_type=jnp.float32) + bk_ref[...]
    qh, ql = _hi_lo(q)
    kh, kl = _hi_lo(k)
    tb = t.astype(jnp.bfloat16)
    qdh_ref[0, 0] = qh
    qdh_ref[0, 1] = qh
    qdl_ref[0, 0] = ql
    qdl_ref[0, 1] = ql
    kh_ref[0] = kh
    kl_ref[0] = kl
    td_ref[0, 0] = tb
    td_ref[0, 1] = tb


def _corr_body(kh_ref, kl_ref, qdh_ref, qdl_ref, mv_ref):
    i = pl.program_id(1)
    t0 = i * TB

    @pl.when(i == 0)
    def _():
        mv_ref[...] = jnp.zeros((1, 1, L), jnp.float32)

    kh = kh_ref[0]
    kl = kl_ref[0]
    khl = jnp.concatenate([kh, kl], axis=0)
    dims = (((1,), (1,)), ((), ()))
    for tt in range(L // TW):
        tau0 = tt * TW
        s0 = pl.multiple_of(lax.rem(t0 + tau0, L), TB)
        qfh = qdh_ref[0, pl.ds(s0, KF), :]
        qfl = qdl_ref[0, pl.ds(s0, KF), :]
        ahl = lax.dot_general(khl, qfh, dims, preferred_element_type=jnp.float32)
        a = (ahl[:TB] + ahl[TB:]
             + lax.dot_general(kh, qfl, dims, preferred_element_type=jnp.float32))
        x = pltpu.roll(a, KF - TB + 1, axis=1, stride=1, stride_axis=0)
        s = jnp.sum(x, axis=0, keepdims=True)
        mv_ref[0, 0:1, tau0:tau0 + TW] += s[0:1, :TW] * (1.0 / D)


def _stats_body(mv_ref, idx_ref, tmp_ref):
    mv = mv_ref[...].reshape(BN, L)
    m = (mv[0:1, :] + mv[1:2, :]) * 0.5
    iota = lax.broadcasted_iota(jnp.int32, (1, L), 1)
    ws = []
    for i in range(NR):
        amax = jnp.max(m)
        sel_first = iota == jnp.min(jnp.where(m == amax, iota, L))
        idx_ref[0, i] = jnp.sum(jnp.where(sel_first, iota, 0))
        ws.append((jnp.sum(jnp.where(sel_first, mv[0:1, :], 0.0)),
                   jnp.sum(jnp.where(sel_first, mv[1:2, :], 0.0))))
        m = jnp.where(sel_first, _NEG, m)
    lane = lax.broadcasted_iota(jnp.int32, (BN, 128), 1)
    row = lax.broadcasted_iota(jnp.int32, (BN, 128), 0)
    wvec = jnp.zeros((BN, 128), jnp.float32)
    for i, (w0, w1) in enumerate(ws):
        wvec += jnp.where(lane == i, jnp.where(row == 0, w0, w1), 0.0)
    wvec = wvec * (1.0 / math.sqrt(D))
    mask = lane < NR
    mx = jnp.max(jnp.where(mask, wvec, _NEG), axis=1, keepdims=True)
    e = jnp.where(mask, jnp.exp(wvec - mx), 0.0)
    tmp_ref[...] = (e / jnp.sum(e, axis=1, keepdims=True)).reshape(BN, 1, 128)


def _rolled_rows(td_ref, idx, l0):
    a8 = pl.multiple_of(l0 + (idx // 8) * 8, 8)
    r = idx % 8
    x = td_ref[0, pl.ds(a8, TB + 8), :]
    return pltpu.roll(x, lax.rem(TB + 8 - r, TB + 8), axis=0)[:TB]


def _roll_body(td_ref, idx_ref, tmp_ref, wv_ref, bv_ref, o1_ref, o2_ref):
    j = pl.program_id(1)
    l0 = j * TB
    i0 = idx_ref[0, 0]
    i4 = idx_ref[0, NR - 1]
    wvb = wv_ref[...].astype(jnp.bfloat16)
    dims = (((1,), (1,)), ((), ()))
    for idx, col, o_ref in ((i0, 0, o1_ref), (i4, NR - 1, o2_ref)):
        rows = _rolled_rows(td_ref, idx, l0)
        v = lax.dot_general(rows, wvb, dims,
                            preferred_element_type=jnp.float32) + bv_ref[...]
        o_ref[0] = v * tmp_ref[0, 0:1, col:col + 1]


def kernel(series, text_series, Wq, bq, Wk, bk, Wv, bv):
    f32 = jnp.float32
    nb = L // TB
    arb = pltpu.CompilerParams(dimension_semantics=("arbitrary", "arbitrary"))

    qdh, qdl, kh, kl, td = pl.pallas_call(
        _proj_body,
        grid=(BN, nb),
        in_specs=[
            pl.BlockSpec((1, TB, D), lambda b, i: (b, i, 0)),
            pl.BlockSpec((1, TB, D), lambda b, i: (b, i, 0)),
            pl.BlockSpec((D, D), lambda b, i: (0, 0)),
            pl.BlockSpec((1, D), lambda b, i: (0, 0)),
            pl.BlockSpec((D, D), lambda b, i: (0, 0)),
            pl.BlockSpec((1, D), lambda b, i: (0, 0)),
        ],
        out_specs=[
            pl.BlockSpec((1, 2, TB, D), lambda b, i: (b, 0, i, 0)),
            pl.BlockSpec((1, 2, TB, D), lambda b, i: (b, 0, i, 0)),
            pl.BlockSpec((1, TB, D), lambda b, i: (b, i, 0)),
            pl.BlockSpec((1, TB, D), lambda b, i: (b, i, 0)),
            pl.BlockSpec((1, 2, TB, D), lambda b, i: (b, 0, i, 0)),
        ],
        out_shape=[
            jax.ShapeDtypeStruct((BN, 2, L, D), jnp.bfloat16),
            jax.ShapeDtypeStruct((BN, 2, L, D), jnp.bfloat16),
            jax.ShapeDtypeStruct((BN, L, D), jnp.bfloat16),
            jax.ShapeDtypeStruct((BN, L, D), jnp.bfloat16),
            jax.ShapeDtypeStruct((BN, 2, L, D), jnp.bfloat16),
        ],
        compiler_params=arb,
    )(series, text_series, Wq, bq.reshape(1, D), Wk, bk.reshape(1, D))

    qdh = qdh.reshape(BN, 2 * L, D)
    qdl = qdl.reshape(BN, 2 * L, D)
    td = td.reshape(BN, 2 * L, D)

    khr = kh[:, ::-1, :]
    klr = kl[:, ::-1, :]

    mv = pl.pallas_call(
        _corr_body,
        grid=(BN, nb),
        in_specs=[
            pl.BlockSpec((1, TB, D), lambda b, i: (b, nb - 1 - i, 0)),
            pl.BlockSpec((1, TB, D), lambda b, i: (b, nb - 1 - i, 0)),
            pl.BlockSpec((1, L + KF, D), lambda b, i: (b, 0, 0)),
            pl.BlockSpec((1, L + KF, D), lambda b, i: (b, 0, 0)),
        ],
        out_specs=pl.BlockSpec((1, 1, L), lambda b, i: (b, 0, 0)),
        out_shape=jax.ShapeDtypeStruct((BN, 1, L), f32),
        compiler_params=arb,
    )(khr, klr, qdh, qdl)

    idxs, tmp_corr = pl.pallas_call(
        _stats_body,
        in_specs=[pl.BlockSpec((BN, 1, L), lambda: (0, 0, 0))],
        out_specs=[
            pl.BlockSpec(memory_space=pltpu.SMEM),
            pl.BlockSpec((BN, 1, 128), lambda: (0, 0, 0)),
        ],
        out_shape=[
            jax.ShapeDtypeStruct((1, 8), jnp.int32),
            jax.ShapeDtypeStruct((BN, 1, 128), f32),
        ],
    )(mv)

    o1, o2 = pl.pallas_call(
        _roll_body,
        grid=(BN, nb),
        in_specs=[
            pl.BlockSpec((1, 2 * L, D), lambda b, j: (b, 0, 0)),
            pl.BlockSpec(memory_space=pltpu.SMEM),
            pl.BlockSpec((1, 1, 128), lambda b, j: (b, 0, 0)),
            pl.BlockSpec((D, D), lambda b, j: (0, 0)),
            pl.BlockSpec((1, D), lambda b, j: (0, 0)),
        ],
        out_specs=[
            pl.BlockSpec((1, TB, D), lambda b, j: (b, j, 0)),
            pl.BlockSpec((1, TB, D), lambda b, j: (b, j, 0)),
        ],
        out_shape=[
            jax.ShapeDtypeStruct((BN, L, D), f32),
            jax.ShapeDtypeStruct((BN, L, D), f32),
        ],
        compiler_params=arb,
    )(td, idxs, tmp_corr, Wv, bv.reshape(1, D))

    return (o1, o2)


# speedup vs baseline: 2.1433x; 2.1433x over previous
import math

import jax
import jax.numpy as jnp
from jax import lax
from jax.experimental import pallas as pl
from jax.experimental.pallas import tpu as pltpu

D = 768
L = 4096
BN = 2
NR = 5
TB = 256
TW = 2048
KF = TW + TB
_NEG = -3.0e38


def _hi_lo(x):
    hi = x.astype(jnp.bfloat16)
    lo = (x - hi.astype(jnp.float32)).astype(jnp.bfloat16)
    return hi, lo


def _proj_body(s_ref, t_ref, wq_ref, bq_ref, wk_ref, bk_ref,
               qdh_ref, qdl_ref, kh_ref, kl_ref, td_ref):
    s = s_ref[0]
    t = t_ref[0]
    dims = (((1,), (1,)), ((), ()))
    q = lax.dot_general(s, wq_ref[...], dims,
                        preferred_element_type=jnp.float32) + bq_ref[...]
    k = lax.dot_general(t, wk_ref[...], dims,
                        preferred_element_type=jnp.float32) + bk_ref[...]
    qh, ql = _hi_lo(q)
    kh, kl = _hi_lo(k)
    tb = t.astype(jnp.bfloat16)
    qdh_ref[0, 0] = qh
    qdh_ref[0, 1] = qh
    qdl_ref[0, 0] = ql
    qdl_ref[0, 1] = ql
    kh_ref[0] = kh
    kl_ref[0] = kl
    td_ref[0, 0] = tb
    td_ref[0, 1] = tb


def _corr_body(kh_ref, kl_ref, qdh_ref, qdl_ref, mv_ref):
    i = pl.program_id(1)
    t0 = i * TB

    @pl.when(i == 0)
    def _():
        mv_ref[...] = jnp.zeros((1, 1, L), jnp.float32)

    kh = kh_ref[0]
    kl = kl_ref[0]
    khl = jnp.concatenate([kh, kl], axis=0)
    dims = (((1,), (1,)), ((), ()))
    for tt in range(L // TW):
        tau0 = tt * TW
        s0 = pl.multiple_of(lax.rem(t0 + tau0, L), TB)
        qfh = qdh_ref[0, pl.ds(s0, KF), :]
        qfl = qdl_ref[0, pl.ds(s0, KF), :]
        ahl = lax.dot_general(khl, qfh, dims, preferred_element_type=jnp.float32)
        a = (ahl[:TB] + ahl[TB:]
             + lax.dot_general(kh, qfl, dims, preferred_element_type=jnp.float32))
        x = a
        m = TB // 2
        while m >= 1:
            x = x[:m] + pltpu.roll(x[m:], KF - m, axis=1)
            m //= 2
        mv_ref[0, 0:1, tau0:tau0 + TW] += x[0:1, :TW] * (1.0 / D)


def _stats_body(mv_ref, idx_ref, tmp_ref):
    mv = mv_ref[...].reshape(BN, L)
    m = (mv[0:1, :] + mv[1:2, :]) * 0.5
    iota = lax.broadcasted_iota(jnp.int32, (1, L), 1)
    ws = []
    for i in range(NR):
        amax = jnp.max(m)
        sel_first = iota == jnp.min(jnp.where(m == amax, iota, L))
        idx_ref[0, i] = jnp.sum(jnp.where(sel_first, iota, 0))
        ws.append((jnp.sum(jnp.where(sel_first, mv[0:1, :], 0.0)),
                   jnp.sum(jnp.where(sel_first, mv[1:2, :], 0.0))))
        m = jnp.where(sel_first, _NEG, m)
    lane = lax.broadcasted_iota(jnp.int32, (BN, 128), 1)
    row = lax.broadcasted_iota(jnp.int32, (BN, 128), 0)
    wvec = jnp.zeros((BN, 128), jnp.float32)
    for i, (w0, w1) in enumerate(ws):
        wvec += jnp.where(lane == i, jnp.where(row == 0, w0, w1), 0.0)
    wvec = wvec * (1.0 / math.sqrt(D))
    mask = lane < NR
    mx = jnp.max(jnp.where(mask, wvec, _NEG), axis=1, keepdims=True)
    e = jnp.where(mask, jnp.exp(wvec - mx), 0.0)
    tmp_ref[...] = (e / jnp.sum(e, axis=1, keepdims=True)).reshape(BN, 1, 128)


def _rolled_rows(td_ref, idx, l0):
    a8 = pl.multiple_of(l0 + (idx // 8) * 8, 8)
    r = idx % 8
    x = td_ref[0, pl.ds(a8, TB + 8), :]
    return pltpu.roll(x, lax.rem(TB + 8 - r, TB + 8), axis=0)[:TB]


def _roll_body(td_ref, idx_ref, tmp_ref, wv_ref, bv_ref, o1_ref, o2_ref):
    j = pl.program_id(1)
    l0 = j * TB
    i0 = idx_ref[0, 0]
    i4 = idx_ref[0, NR - 1]
    wvb = wv_ref[...].astype(jnp.bfloat16)
    dims = (((1,), (1,)), ((), ()))
    for idx, col, o_ref in ((i0, 0, o1_ref), (i4, NR - 1, o2_ref)):
        rows = _rolled_rows(td_ref, idx, l0)
        v = lax.dot_general(rows, wvb, dims,
                            preferred_element_type=jnp.float32) + bv_ref[...]
        o_ref[0] = v * tmp_ref[0, 0:1, col:col + 1]


def kernel(series, text_series, Wq, bq, Wk, bk, Wv, bv):
    f32 = jnp.float32
    nb = L // TB
    TP = 512
    npb = L // TP
    arb = pltpu.CompilerParams(dimension_semantics=("arbitrary", "arbitrary"))

    qdh, qdl, kh, kl, td = pl.pallas_call(
        _proj_body,
        grid=(BN, npb),
        in_specs=[
            pl.BlockSpec((1, TP, D), lambda b, i: (b, i, 0)),
            pl.BlockSpec((1, TP, D), lambda b, i: (b, i, 0)),
            pl.BlockSpec((D, D), lambda b, i: (0, 0)),
            pl.BlockSpec((1, D), lambda b, i: (0, 0)),
            pl.BlockSpec((D, D), lambda b, i: (0, 0)),
            pl.BlockSpec((1, D), lambda b, i: (0, 0)),
        ],
        out_specs=[
            pl.BlockSpec((1, 2, TP, D), lambda b, i: (b, 0, i, 0)),
            pl.BlockSpec((1, 2, TP, D), lambda b, i: (b, 0, i, 0)),
            pl.BlockSpec((1, TP, D), lambda b, i: (b, i, 0)),
            pl.BlockSpec((1, TP, D), lambda b, i: (b, i, 0)),
            pl.BlockSpec((1, 2, TP, D), lambda b, i: (b, 0, i, 0)),
        ],
        out_shape=[
            jax.ShapeDtypeStruct((BN, 2, L, D), jnp.bfloat16),
            jax.ShapeDtypeStruct((BN, 2, L, D), jnp.bfloat16),
            jax.ShapeDtypeStruct((BN, L, D), jnp.bfloat16),
            jax.ShapeDtypeStruct((BN, L, D), jnp.bfloat16),
            jax.ShapeDtypeStruct((BN, 2, L, D), jnp.bfloat16),
        ],
        compiler_params=arb,
    )(series, text_series, Wq, bq.reshape(1, D), Wk, bk.reshape(1, D))

    qdh = qdh.reshape(BN, 2 * L, D)
    qdl = qdl.reshape(BN, 2 * L, D)
    td = td.reshape(BN, 2 * L, D)

    mv = pl.pallas_call(
        _corr_body,
        grid=(BN, nb),
        in_specs=[
            pl.BlockSpec((1, TB, D), lambda b, i: (b, i, 0)),
            pl.BlockSpec((1, TB, D), lambda b, i: (b, i, 0)),
            pl.BlockSpec((1, L + KF, D), lambda b, i: (b, 0, 0)),
            pl.BlockSpec((1, L + KF, D), lambda b, i: (b, 0, 0)),
        ],
        out_specs=pl.BlockSpec((1, 1, L), lambda b, i: (b, 0, 0)),
        out_shape=jax.ShapeDtypeStruct((BN, 1, L), f32),
        compiler_params=arb,
    )(kh, kl, qdh, qdl)

    idxs, tmp_corr = pl.pallas_call(
        _stats_body,
        in_specs=[pl.BlockSpec((BN, 1, L), lambda: (0, 0, 0))],
        out_specs=[
            pl.BlockSpec(memory_space=pltpu.SMEM),
            pl.BlockSpec((BN, 1, 128), lambda: (0, 0, 0)),
        ],
        out_shape=[
            jax.ShapeDtypeStruct((1, 8), jnp.int32),
            jax.ShapeDtypeStruct((BN, 1, 128), f32),
        ],
    )(mv)

    o1, o2 = pl.pallas_call(
        _roll_body,
        grid=(BN, nb),
        in_specs=[
            pl.BlockSpec((1, 2 * L, D), lambda b, j: (b, 0, 0)),
            pl.BlockSpec(memory_space=pltpu.SMEM),
            pl.BlockSpec((1, 1, 128), lambda b, j: (b, 0, 0)),
            pl.BlockSpec((D, D), lambda b, j: (0, 0)),
            pl.BlockSpec((1, D), lambda b, j: (0, 0)),
        ],
        out_specs=[
            pl.BlockSpec((1, TB, D), lambda b, j: (b, j, 0)),
            pl.BlockSpec((1, TB, D), lambda b, j: (b, j, 0)),
        ],
        out_shape=[
            jax.ShapeDtypeStruct((BN, L, D), f32),
            jax.ShapeDtypeStruct((BN, L, D), f32),
        ],
        compiler_params=arb,
    )(td, idxs, tmp_corr, Wv, bv.reshape(1, D))

    return (o1, o2)
